# Initial kernel scaffold; baseline (speedup 1.0000x reference)
import jax
import jax.numpy as jnp
from jax.experimental import pallas as pl


def kernel(x, edge_index, node_label, node_index, W1, b1, W2, b2, W3, b3, Wout, bout):
    raise NotImplementedError("write your pallas kernel here")


# speedup vs baseline: 4.5091x; 4.5091x over previous
import functools

import jax
import jax.numpy as jnp
from jax import lax
from jax.experimental import pallas as pl
from jax.experimental.pallas import tpu as pltpu
from jax.experimental.pallas import tpu_sc as plsc

N = 10000
E = 320000
D = 128
C = 3
B = 5000

NC = 2
NS = 16
NW = NC * NS

CH = 128
CPW = 79
EPW = CH * CPW
EPAD = EPW * NW
NPAD = 10112
ZR = NPAD // NS
WB = 624

GCH = 80
GPW = 2 * GCH
BPAD = GPW * NW

BM = 1000


def _linear_body(x_ref, w_ref, b_ref, o_ref):
    o_ref[...] = (
        jnp.dot(x_ref[...], w_ref[...], preferred_element_type=jnp.float32)
        + b_ref[...]
    )


def _linear(x, W, b2d):
    return pl.pallas_call(
        _linear_body,
        grid=(N // BM,),
        in_specs=[
            pl.BlockSpec((BM, D), lambda i: (i, 0)),
            pl.BlockSpec((D, D), lambda i: (0, 0)),
            pl.BlockSpec((1, D), lambda i: (0, 0)),
        ],
        out_specs=pl.BlockSpec((BM, D), lambda i: (i, 0)),
        out_shape=jax.ShapeDtypeStruct((N, D), jnp.float32),
    )(x, W, b2d)


def _combine_linear_body(p0_ref, p1_ref, h_ref, w_ref, b_ref, o_ref):
    act = jnp.maximum(p0_ref[...] + p1_ref[...] + h_ref[...], 0.0)
    o_ref[...] = (
        jnp.dot(act, w_ref[...], preferred_element_type=jnp.float32)
        + b_ref[...]
    )


def _combine_linear(p, h, W, b2d):
    return pl.pallas_call(
        _combine_linear_body,
        grid=(N // BM,),
        in_specs=[
            pl.BlockSpec((BM, D), lambda i: (i, 0)),
            pl.BlockSpec((BM, D), lambda i: (i + N // BM, 0)),
            pl.BlockSpec((BM, D), lambda i: (i, 0)),
            pl.BlockSpec((D, D), lambda i: (0, 0)),
            pl.BlockSpec((1, D), lambda i: (0, 0)),
        ],
        out_specs=pl.BlockSpec((BM, D), lambda i: (i, 0)),
        out_shape=jax.ShapeDtypeStruct((N, D), jnp.float32),
    )(p, p, h, W, b2d)


def _head_body(p0_ref, p1_ref, h_ref, w_ref, b_ref, xe_ref, out_ref, yp_ref):
    xe = jnp.maximum(p0_ref[...] + p1_ref[...] + h_ref[...], 0.0)
    xe_ref[...] = xe
    logits = (
        jnp.dot(xe, w_ref[...], preferred_element_type=jnp.float32)
        + b_ref[...]
    )
    out_ref[...] = logits
    m = jnp.max(logits, axis=1, keepdims=True)
    idx = lax.broadcasted_iota(jnp.int32, logits.shape, 1)
    yp_ref[...] = jnp.min(jnp.where(logits == m, idx, D), axis=1,
                          keepdims=True)


def _head(p, h, Wp, bp2d):
    return pl.pallas_call(
        _head_body,
        grid=(N // BM,),
        in_specs=[
            pl.BlockSpec((BM, D), lambda i: (i, 0)),
            pl.BlockSpec((BM, D), lambda i: (i + N // BM, 0)),
            pl.BlockSpec((BM, D), lambda i: (i, 0)),
            pl.BlockSpec((D, D), lambda i: (0, 0)),
            pl.BlockSpec((1, D), lambda i: (0, 0)),
        ],
        out_specs=[
            pl.BlockSpec((BM, D), lambda i: (i, 0)),
            pl.BlockSpec((BM, D), lambda i: (i, 0)),
            pl.BlockSpec((BM, 1), lambda i: (i, 0)),
        ],
        out_shape=[
            jax.ShapeDtypeStruct((N, D), jnp.float32),
            jax.ShapeDtypeStruct((N, D), jnp.float32),
            jax.ShapeDtypeStruct((N, 1), jnp.int32),
        ],
    )(p, p, h, Wp, bp2d)


def _segsum(h, src3, dst3, zrows):
    mesh = plsc.VectorSubcoreMesh(core_axis_name="c", subcore_axis_name="s")

    @functools.partial(
        pl.kernel,
        out_type=jax.ShapeDtypeStruct((2 * N, D), jnp.float32),
        mesh=mesh,
        scratch_types=[
            pltpu.VMEM((CPW, CH), jnp.int32),
            pltpu.VMEM((CPW, CH), jnp.int32),
            pltpu.VMEM((CH, D), jnp.float32),
            pltpu.VMEM_SHARED((NPAD, D), jnp.float32),
            pltpu.SemaphoreType.DMA,
        ],
    )
    def k(h_hbm, src_hbm, dst_hbm, z_hbm, out_hbm, src_v, dst_v, rows_v,
          agg_sh, sem):
        cid = lax.axis_index("c")
        sid = lax.axis_index("s")
        wid = sid * NC + cid
        pltpu.sync_copy(z_hbm, agg_sh.at[pl.ds(sid * ZR, ZR)])
        pltpu.sync_copy(src_hbm.at[wid], src_v)
        pltpu.sync_copy(dst_hbm.at[wid], dst_v)
        plsc.subcore_barrier()

        def body(j, carry):
            pltpu.async_copy(h_hbm.at[src_v.at[j]], rows_v, sem).wait()
            pltpu.sync_copy(rows_v, agg_sh.at[dst_v.at[j]], add=True)
            return carry

        lax.fori_loop(0, CPW, body, 0)
        plsc.subcore_barrier()
        pltpu.sync_copy(
            agg_sh.at[pl.ds(sid * WB, WB)],
            out_hbm.at[pl.ds(cid * N + sid * WB, WB)],
        )

        @pl.when(sid == NS - 1)
        def _tail():
            pltpu.sync_copy(
                agg_sh.at[pl.ds(NS * WB, N - NS * WB)],
                out_hbm.at[pl.ds(cid * N + NS * WB, N - NS * WB)],
            )

    return k(h, src3, dst3, zrows)


def _gather_outputs(logits, ni3):
    mesh = plsc.VectorSubcoreMesh(core_axis_name="c", subcore_axis_name="s")

    @functools.partial(
        pl.kernel,
        out_type=jax.ShapeDtypeStruct((BPAD, D), jnp.float32),
        mesh=mesh,
        scratch_types=[
            pltpu.VMEM((2, GCH), jnp.int32),
            pltpu.VMEM((GCH, D), jnp.float32),
            pltpu.SemaphoreType.DMA,
        ],
    )
    def k(log_hbm, ni_hbm, rows_hbm, ni_v, rows_v, sem):
        cid = lax.axis_index("c")
        sid = lax.axis_index("s")
        wid = sid * NC + cid
        pltpu.sync_copy(ni_hbm.at[wid], ni_v)

        def chunk(j, carry):
            pltpu.async_copy(log_hbm.at[ni_v.at[j]], rows_v, sem).wait()
            pltpu.sync_copy(
                rows_v, rows_hbm.at[pl.ds(wid * GPW + j * GCH, GCH)]
            )
            return carry

        lax.fori_loop(0, 2, chunk, 0)

    return k(logits, ni3)


def _row_argmax_body(r_ref, yv_ref):
    logits = r_ref[...]
    m = jnp.max(logits, axis=1, keepdims=True)
    idx = lax.broadcasted_iota(jnp.int32, logits.shape, 1)
    yv_ref[...] = jnp.min(jnp.where(logits == m, idx, D), axis=1,
                          keepdims=True)


def _row_argmax(rows):
    return pl.pallas_call(
        _row_argmax_body,
        grid=(BPAD // 640,),
        in_specs=[pl.BlockSpec((640, D), lambda i: (i, 0))],
        out_specs=pl.BlockSpec((640, 1), lambda i: (i, 0)),
        out_shape=jax.ShapeDtypeStruct((BPAD, 1), jnp.int32),
    )(rows)


def kernel(x, edge_index, node_label, node_index, W1, b1, W2, b2, W3, b3,
           Wout, bout):
    del node_label

    src = edge_index[0]
    dst = edge_index[1]
    pad = EPAD - E
    src3 = jnp.concatenate([src, jnp.zeros((pad,), jnp.int32)]).reshape(
        NW, CPW, CH)
    dst3 = jnp.concatenate([dst, jnp.full((pad,), N, jnp.int32)]).reshape(
        NW, CPW, CH)
    zrows = jnp.zeros((ZR, D), jnp.float32)

    b1r = b1.reshape(1, D)
    b2r = b2.reshape(1, D)
    b3r = b3.reshape(1, D)
    Wp = jnp.zeros((D, D), jnp.float32).at[:, :C].set(Wout)
    bp = jnp.full((D,), -1e30, jnp.float32).at[:C].set(bout).reshape(1, D)

    h1 = _linear(x, W1, b1r)
    p1 = _segsum(h1, src3, dst3, zrows)
    h2 = _combine_linear(p1, h1, W2, b2r)
    p2 = _segsum(h2, src3, dst3, zrows)
    h3 = _combine_linear(p2, h2, W3, b3r)
    p3 = _segsum(h3, src3, dst3, zrows)
    x_embed, logits_pad, yp2d = _head(p3, h3, Wp, bp)
    ypred = yp2d.reshape(N)

    ni3 = jnp.concatenate(
        [node_index, jnp.zeros((BPAD - B,), jnp.int32)]).reshape(NW, 2, GCH)
    rows = _gather_outputs(logits_pad, ni3)
    yv = _row_argmax(rows)

    node_output = rows[:B, :C]
    y_nodepred = yv.reshape(BPAD)[:B]
    return (x_embed, node_output, ypred, y_nodepred)
